# baseline (speedup 1.0000x reference)
import functools

import jax
import jax.numpy as jnp
from jax import lax
from jax.experimental import pallas as pl
from jax.experimental.pallas import tpu as pltpu
from jax.experimental.pallas import tpu_sc as plsc

B = 1024
CTX = 50
EMB = 64
VOC = 100000

NC = 2
NS = 16
NW = NC * NS
N_IDX = B * CTX
IDX_PER_W = N_IDX // NW
PAIR = 2 * EMB
CHUNK = 400
N_CHUNKS = IDX_PER_W // CHUNK
MAX_DESC = 128


def _fire_gathers(table_hbm, idx_v, buf, base_off, sem):
    descs = []
    off = 0
    while off < CHUNK:
        n = min(MAX_DESC, CHUNK - off)
        descs.append(
            pltpu.async_copy(
                table_hbm.at[idx_v.at[pl.ds(base_off + off, n)]],
                buf.at[pl.ds(off, n), :],
                sem,
            )
        )
        off += n
    return descs


def _gather_body(idx_hbm, table_hbm, out_hbm, idx_v, buf0, buf1,
                 sg0, sg1, so0, so1):
    wid = lax.axis_index("s") * NC + lax.axis_index("c")
    base = wid * IDX_PER_W
    pltpu.sync_copy(idx_hbm.at[pl.ds(base, IDX_PER_W)], idx_v)

    bufs = (buf0, buf1)
    gsems = (sg0, sg1)
    osems = (so0, so1)
    gd = [None] * N_CHUNKS
    od = [None] * N_CHUNKS
    for c in range(N_CHUNKS):
        p = c % 2
        if c >= 2:
            od[c - 2].wait()
        gd[c] = _fire_gathers(table_hbm, idx_v, bufs[p], c * CHUNK, gsems[p])
        if c >= 1:
            q = (c - 1) % 2
            for d in gd[c - 1]:
                d.wait()
            od[c - 1] = pltpu.async_copy(
                bufs[q],
                out_hbm.at[pl.ds(base + (c - 1) * CHUNK, CHUNK)],
                osems[q],
            )
    last = N_CHUNKS - 1
    for d in gd[last]:
        d.wait()
    od[last] = pltpu.async_copy(
        bufs[last % 2],
        out_hbm.at[pl.ds(base + last * CHUNK, CHUNK)],
        osems[last % 2],
    )
    if N_CHUNKS >= 2:
        od[last - 1].wait()
    od[last].wait()


_gather_sc = functools.partial(
    pl.kernel,
    out_type=jax.ShapeDtypeStruct((N_IDX, PAIR), jnp.float32),
    mesh=plsc.VectorSubcoreMesh(
        core_axis_name="c", subcore_axis_name="s", num_cores=NC,
        num_subcores=NS,
    ),
    scratch_types=[
        pltpu.VMEM((IDX_PER_W,), jnp.int32),
        pltpu.VMEM((CHUNK, PAIR), jnp.float32),
        pltpu.VMEM((CHUNK, PAIR), jnp.float32),
        pltpu.SemaphoreType.DMA,
        pltpu.SemaphoreType.DMA,
        pltpu.SemaphoreType.DMA,
        pltpu.SemaphoreType.DMA,
    ],
)(_gather_body)


B_POOL = 256


def _pool_body(g_ref, par_ref, o_ref):
    g = g_ref[...]
    h0 = g[:, :, :EMB]
    h1 = g[:, :, EMB:]
    sel = jnp.where(par_ref[...][:, :, None] > 0, h1, h0)
    o_ref[...] = jnp.mean(sel, axis=1)


def _pool_tc(gathered, parf):
    return pl.pallas_call(
        _pool_body,
        grid=(B // B_POOL,),
        in_specs=[
            pl.BlockSpec((B_POOL, CTX, PAIR), lambda i: (i, 0, 0)),
            pl.BlockSpec((B_POOL, CTX), lambda i: (i, 0)),
        ],
        out_specs=pl.BlockSpec((B_POOL, EMB), lambda i: (i, 0)),
        out_shape=jax.ShapeDtypeStruct((B, EMB), jnp.float32),
    )(gathered, parf)


N_BLK = 2048


def _mm_body(p_ref, w_ref, b_ref, o_ref):
    o_ref[...] = (
        jnp.dot(p_ref[...], w_ref[...], preferred_element_type=jnp.float32)
        + b_ref[...]
    )


def _matmul_tc(pooled, W, b2d):
    n_blocks = pl.cdiv(VOC, N_BLK)
    return pl.pallas_call(
        _mm_body,
        grid=(n_blocks,),
        in_specs=[
            pl.BlockSpec((B, EMB), lambda i: (0, 0)),
            pl.BlockSpec((EMB, N_BLK), lambda i: (0, i)),
            pl.BlockSpec((1, N_BLK), lambda i: (0, i)),
        ],
        out_specs=pl.BlockSpec((B, N_BLK), lambda i: (0, i)),
        out_shape=jax.ShapeDtypeStruct((B, VOC), jnp.float32),
        compiler_params=pltpu.CompilerParams(
            dimension_semantics=("arbitrary",),
        ),
    )(pooled, W, b2d)


@jax.jit
def kernel(inpt, table, W, b):
    idx_flat = inpt.astype(jnp.int32).reshape(N_IDX)
    idx_pair = idx_flat >> 1
    parf = (inpt.astype(jnp.int32) & 1).reshape(B, CTX)
    table_pairs = table.reshape(VOC // 2, PAIR)
    g = _gather_sc(idx_pair, table_pairs)
    pooled = _pool_tc(g.reshape(B, CTX, PAIR), parf)
    return _matmul_tc(pooled, W, b.reshape(1, VOC))


# speedup vs baseline: 1.0038x; 1.0038x over previous
import functools

import jax
import jax.numpy as jnp
from jax import lax
from jax.experimental import pallas as pl
from jax.experimental.pallas import tpu as pltpu
from jax.experimental.pallas import tpu_sc as plsc

B = 1024
CTX = 50
EMB = 64
VOC = 100000

NC = 2
NS = 16
NW = NC * NS
N_IDX = B * CTX
IDX_PER_W = N_IDX // NW
PAIR = 2 * EMB
CHUNK = 400
N_CHUNKS = IDX_PER_W // CHUNK
MAX_DESC = 128


def _fire_gathers(table_hbm, idx_v, buf, base_off, sem):
    descs = []
    off = 0
    while off < CHUNK:
        n = min(MAX_DESC, CHUNK - off)
        descs.append(
            pltpu.async_copy(
                table_hbm.at[idx_v.at[pl.ds(base_off + off, n)]],
                buf.at[pl.ds(off, n), :],
                sem,
            )
        )
        off += n
    return descs


def _gather_body(idx_hbm, table_hbm, out_hbm, idx_v, buf0, buf1,
                 sg0, sg1, so0, so1):
    wid = lax.axis_index("s") * NC + lax.axis_index("c")
    base = wid * IDX_PER_W
    pltpu.sync_copy(idx_hbm.at[pl.ds(base, IDX_PER_W)], idx_v)

    bufs = (buf0, buf1)
    gsems = (sg0, sg1)
    osems = (so0, so1)
    gd = [None] * N_CHUNKS
    od = [None] * N_CHUNKS
    for c in range(N_CHUNKS):
        p = c % 2
        if c >= 2:
            od[c - 2].wait()
        gd[c] = _fire_gathers(table_hbm, idx_v, bufs[p], c * CHUNK, gsems[p])
        if c >= 1:
            q = (c - 1) % 2
            for d in gd[c - 1]:
                d.wait()
            od[c - 1] = pltpu.async_copy(
                bufs[q],
                out_hbm.at[pl.ds(base + (c - 1) * CHUNK, CHUNK)],
                osems[q],
            )
    last = N_CHUNKS - 1
    for d in gd[last]:
        d.wait()
    od[last] = pltpu.async_copy(
        bufs[last % 2],
        out_hbm.at[pl.ds(base + last * CHUNK, CHUNK)],
        osems[last % 2],
    )
    if N_CHUNKS >= 2:
        od[last - 1].wait()
    od[last].wait()


_gather_sc = functools.partial(
    pl.kernel,
    out_type=jax.ShapeDtypeStruct((N_IDX, PAIR), jnp.float32),
    mesh=plsc.VectorSubcoreMesh(
        core_axis_name="c", subcore_axis_name="s", num_cores=NC,
        num_subcores=NS,
    ),
    scratch_types=[
        pltpu.VMEM((IDX_PER_W,), jnp.int32),
        pltpu.VMEM((CHUNK, PAIR), jnp.float32),
        pltpu.VMEM((CHUNK, PAIR), jnp.float32),
        pltpu.SemaphoreType.DMA,
        pltpu.SemaphoreType.DMA,
        pltpu.SemaphoreType.DMA,
        pltpu.SemaphoreType.DMA,
    ],
)(_gather_body)


B_POOL = 256


def _pool_body(g_ref, par_ref, o_ref):
    g = g_ref[...]
    h0 = g[:, :, :EMB]
    h1 = g[:, :, EMB:]
    sel = jnp.where(par_ref[...][:, :, None] > 0, h1, h0)
    o_ref[...] = jnp.mean(sel, axis=1)


def _pool_tc(gathered, parf):
    return pl.pallas_call(
        _pool_body,
        grid=(B // B_POOL,),
        in_specs=[
            pl.BlockSpec((B_POOL, CTX, PAIR), lambda i: (i, 0, 0)),
            pl.BlockSpec((B_POOL, CTX), lambda i: (i, 0)),
        ],
        out_specs=pl.BlockSpec((B_POOL, EMB), lambda i: (i, 0)),
        out_shape=jax.ShapeDtypeStruct((B, EMB), jnp.float32),
    )(gathered, parf)


N_BLK = 2048


def _mm_body(p_ref, w_ref, b_ref, o_ref):
    o_ref[...] = (
        jnp.dot(
            p_ref[...].astype(jnp.bfloat16),
            w_ref[...].astype(jnp.bfloat16),
            preferred_element_type=jnp.float32,
        )
        + b_ref[...]
    )


def _matmul_tc(pooled, W, b2d):
    n_blocks = pl.cdiv(VOC, N_BLK)
    return pl.pallas_call(
        _mm_body,
        grid=(n_blocks,),
        in_specs=[
            pl.BlockSpec((B, EMB), lambda i: (0, 0)),
            pl.BlockSpec((EMB, N_BLK), lambda i: (0, i)),
            pl.BlockSpec((1, N_BLK), lambda i: (0, i)),
        ],
        out_specs=pl.BlockSpec((B, N_BLK), lambda i: (0, i)),
        out_shape=jax.ShapeDtypeStruct((B, VOC), jnp.float32),
        compiler_params=pltpu.CompilerParams(
            dimension_semantics=("arbitrary",),
        ),
    )(pooled, W, b2d)


@jax.jit
def kernel(inpt, table, W, b):
    idx_flat = inpt.astype(jnp.int32).reshape(N_IDX)
    idx_pair = idx_flat >> 1
    parf = (inpt.astype(jnp.int32) & 1).reshape(B, CTX)
    table_pairs = table.reshape(VOC // 2, PAIR)
    g = _gather_sc(idx_pair, table_pairs)
    pooled = _pool_tc(g.reshape(B, CTX, PAIR), parf)
    return _matmul_tc(pooled, W, b.reshape(1, VOC))


# speedup vs baseline: 1.9683x; 1.9608x over previous
import functools

import jax
import jax.numpy as jnp
from jax import lax
from jax.experimental import pallas as pl
from jax.experimental.pallas import tpu as pltpu
from jax.experimental.pallas import tpu_sc as plsc

B = 1024
CTX = 50
EMB = 64
VOC = 100000

NC = 2
NS = 16
NW = NC * NS
N_IDX = B * CTX
IDX_PER_W = N_IDX // NW
PAIR = 2 * EMB
CHUNK = 400
N_CHUNKS = IDX_PER_W // CHUNK
MAX_DESC = 128


def _fire_gathers(table_hbm, idx_v, buf, base_off, sem):
    descs = []
    off = 0
    while off < CHUNK:
        n = min(MAX_DESC, CHUNK - off)
        descs.append(
            pltpu.async_copy(
                table_hbm.at[idx_v.at[pl.ds(base_off + off, n)]],
                buf.at[pl.ds(off, n), :],
                sem,
            )
        )
        off += n
    return descs


def _gather_body(idx_hbm, table_hbm, out_hbm, idx_v, buf0, buf1,
                 sg0, sg1, so0, so1):
    wid = lax.axis_index("s") * NC + lax.axis_index("c")
    base = wid * IDX_PER_W
    pltpu.sync_copy(idx_hbm.at[pl.ds(base, IDX_PER_W)], idx_v)

    bufs = (buf0, buf1)
    gsems = (sg0, sg1)
    osems = (so0, so1)
    gd = [None] * N_CHUNKS
    od = [None] * N_CHUNKS
    for c in range(N_CHUNKS):
        p = c % 2
        if c >= 2:
            od[c - 2].wait()
        gd[c] = _fire_gathers(table_hbm, idx_v, bufs[p], c * CHUNK, gsems[p])
        if c >= 1:
            q = (c - 1) % 2
            for d in gd[c - 1]:
                d.wait()
            od[c - 1] = pltpu.async_copy(
                bufs[q],
                out_hbm.at[pl.ds(base + (c - 1) * CHUNK, CHUNK)],
                osems[q],
            )
    last = N_CHUNKS - 1
    for d in gd[last]:
        d.wait()
    od[last] = pltpu.async_copy(
        bufs[last % 2],
        out_hbm.at[pl.ds(base + last * CHUNK, CHUNK)],
        osems[last % 2],
    )
    if N_CHUNKS >= 2:
        od[last - 1].wait()
    od[last].wait()


_gather_sc = functools.partial(
    pl.kernel,
    out_type=jax.ShapeDtypeStruct((N_IDX, PAIR), jnp.float32),
    mesh=plsc.VectorSubcoreMesh(
        core_axis_name="c", subcore_axis_name="s", num_cores=NC,
        num_subcores=NS,
    ),
    scratch_types=[
        pltpu.VMEM((IDX_PER_W,), jnp.int32),
        pltpu.VMEM((CHUNK, PAIR), jnp.float32),
        pltpu.VMEM((CHUNK, PAIR), jnp.float32),
        pltpu.SemaphoreType.DMA,
        pltpu.SemaphoreType.DMA,
        pltpu.SemaphoreType.DMA,
        pltpu.SemaphoreType.DMA,
    ],
)(_gather_body)


B_POOL = 256


def _pool_body(g_ref, par_ref, o_ref):
    g = g_ref[...]
    h0 = g[:, :, :EMB]
    h1 = g[:, :, EMB:]
    sel = jnp.where(par_ref[...][:, :, None] > 0, h1, h0)
    o_ref[...] = jnp.mean(sel, axis=1).T


def _pool_tc(gathered, parf):
    return pl.pallas_call(
        _pool_body,
        grid=(B // B_POOL,),
        in_specs=[
            pl.BlockSpec((B_POOL, CTX, PAIR), lambda i: (i, 0, 0)),
            pl.BlockSpec((B_POOL, CTX), lambda i: (i, 0)),
        ],
        out_specs=pl.BlockSpec((EMB, B_POOL), lambda i: (0, i)),
        out_shape=jax.ShapeDtypeStruct((EMB, B), jnp.float32),
    )(gathered, parf)


N_BLK = 2048


def _mm_body(w_ref, p_ref, b_ref, o_ref):
    o_ref[...] = (
        lax.dot_general(
            w_ref[...].astype(jnp.bfloat16),
            p_ref[...].astype(jnp.bfloat16),
            (((0,), (0,)), ((), ())),
            preferred_element_type=jnp.float32,
        )
        + b_ref[...]
    )


def _matmul_tc(pooledT, W, bcol):
    n_blocks = pl.cdiv(VOC, N_BLK)
    return pl.pallas_call(
        _mm_body,
        grid=(n_blocks,),
        in_specs=[
            pl.BlockSpec((EMB, N_BLK), lambda i: (0, i)),
            pl.BlockSpec((EMB, B), lambda i: (0, 0)),
            pl.BlockSpec((N_BLK, 1), lambda i: (i, 0)),
        ],
        out_specs=pl.BlockSpec((N_BLK, B), lambda i: (i, 0)),
        out_shape=jax.ShapeDtypeStruct((VOC, B), jnp.float32),
        compiler_params=pltpu.CompilerParams(
            dimension_semantics=("arbitrary",),
        ),
    )(W, pooledT, bcol)


@jax.jit
def kernel(inpt, table, W, b):
    idx_flat = inpt.astype(jnp.int32).reshape(N_IDX)
    idx_pair = idx_flat >> 1
    parf = (inpt.astype(jnp.int32) & 1).reshape(B, CTX)
    table_pairs = table.reshape(VOC // 2, PAIR)
    g = _gather_sc(idx_pair, table_pairs)
    pooledT = _pool_tc(g.reshape(B, CTX, PAIR), parf)
    outT = _matmul_tc(pooledT, W, b.reshape(VOC, 1))
    return outT.T


# speedup vs baseline: 2.3027x; 1.1699x over previous
import functools

import jax
import jax.numpy as jnp
from jax import lax
from jax.experimental import pallas as pl
from jax.experimental.pallas import tpu as pltpu
from jax.experimental.pallas import tpu_sc as plsc

B = 1024
CTX = 50
EMB = 64
VOC = 100000

NC = 2
NS = 16
NW = NC * NS
N_IDX = B * CTX
B_PER_W = B // NW
IDX_PER_W = B_PER_W * CTX
MAX_DESC = 128
LANES = 16
EMB_VREGS = EMB // LANES


def _pool_body(idx_hbm, table_hbm, out_hbm, idx_v, rows_v, pooled_v, sem):
    wid = lax.axis_index("s") * NC + lax.axis_index("c")
    base = wid * IDX_PER_W
    pltpu.sync_copy(idx_hbm.at[pl.ds(base, IDX_PER_W)], idx_v)

    descs = []
    off = 0
    while off < IDX_PER_W:
        n = min(MAX_DESC, IDX_PER_W - off)
        descs.append(
            pltpu.async_copy(
                table_hbm.at[idx_v.at[pl.ds(off, n)]],
                rows_v.at[pl.ds(off, n), :],
                sem,
            )
        )
        off += n
    for d in descs:
        d.wait()

    inv_ctx = jnp.float32(1.0 / CTX)

    def elem_body(e, carry):
        def row_body(r, acc):
            row = e * CTX + r
            return tuple(
                acc[j] + rows_v[row, pl.ds(j * LANES, LANES)]
                for j in range(EMB_VREGS)
            )

        acc0 = tuple(
            jnp.zeros((LANES,), jnp.float32) for _ in range(EMB_VREGS)
        )
        acc = lax.fori_loop(0, CTX, row_body, acc0)
        for j in range(EMB_VREGS):
            pooled_v[e, pl.ds(j * LANES, LANES)] = acc[j] * inv_ctx
        return carry

    lax.fori_loop(0, B_PER_W, elem_body, 0)
    pltpu.sync_copy(pooled_v, out_hbm.at[pl.ds(wid * B_PER_W, B_PER_W)])


_pool_sc = functools.partial(
    pl.kernel,
    out_type=jax.ShapeDtypeStruct((B, EMB), jnp.float32),
    mesh=plsc.VectorSubcoreMesh(
        core_axis_name="c", subcore_axis_name="s", num_cores=NC,
        num_subcores=NS,
    ),
    scratch_types=[
        pltpu.VMEM((IDX_PER_W,), jnp.int32),
        pltpu.VMEM((IDX_PER_W, EMB), jnp.float32),
        pltpu.VMEM((B_PER_W, EMB), jnp.float32),
        pltpu.SemaphoreType.DMA,
    ],
    compiler_params=pltpu.CompilerParams(use_tc_tiling_on_sc=False),
)(_pool_body)


N_BLK = 2048


def _mm_body(w_ref, p_ref, b_ref, o_ref):
    o_ref[...] = (
        lax.dot_general(
            w_ref[...].astype(jnp.bfloat16),
            p_ref[...].astype(jnp.bfloat16),
            (((0,), (1,)), ((), ())),
            preferred_element_type=jnp.float32,
        )
        + b_ref[...]
    )


def _matmul_tc(pooled, W, bcol):
    n_blocks = pl.cdiv(VOC, N_BLK)
    return pl.pallas_call(
        _mm_body,
        grid=(n_blocks,),
        in_specs=[
            pl.BlockSpec((EMB, N_BLK), lambda i: (0, i)),
            pl.BlockSpec((B, EMB), lambda i: (0, 0)),
            pl.BlockSpec((N_BLK, 1), lambda i: (i, 0)),
        ],
        out_specs=pl.BlockSpec((N_BLK, B), lambda i: (i, 0)),
        out_shape=jax.ShapeDtypeStruct((VOC, B), jnp.float32),
        compiler_params=pltpu.CompilerParams(
            dimension_semantics=("arbitrary",),
        ),
    )(W, pooled, bcol)


@jax.jit
def kernel(inpt, table, W, b):
    idx_flat = inpt.astype(jnp.int32).reshape(N_IDX)
    pooled = _pool_sc(idx_flat, table)
    outT = _matmul_tc(pooled, W, b.reshape(VOC, 1))
    return outT.T


# speedup vs baseline: 2.8090x; 1.2199x over previous
import functools

import jax
import jax.numpy as jnp
from jax import lax
from jax.experimental import pallas as pl
from jax.experimental.pallas import tpu as pltpu
from jax.experimental.pallas import tpu_sc as plsc

B = 1024
CTX = 50
EMB = 64
VOC = 100000

NC = 2
NS = 16
NW = NC * NS
N_IDX = B * CTX
B_PER_W = B // NW
IDX_PER_W = B_PER_W * CTX
MAX_DESC = 128
LANES = 16
EMB_VREGS = EMB // LANES


def _pool_body(idx_hbm, table_hbm, out_hbm, idx_v, rows_v, pooled_v, sem):
    wid = lax.axis_index("s") * NC + lax.axis_index("c")
    base = wid * IDX_PER_W
    pltpu.sync_copy(idx_hbm.at[pl.ds(base, IDX_PER_W)], idx_v)

    descs = []
    off = 0
    while off < IDX_PER_W:
        n = min(MAX_DESC, IDX_PER_W - off)
        descs.append(
            pltpu.async_copy(
                table_hbm.at[idx_v.at[pl.ds(off, n)]],
                rows_v.at[pl.ds(off, n), :],
                sem,
            )
        )
        off += n
    for d in descs:
        d.wait()

    inv_ctx = jnp.float32(1.0 / CTX)

    def elem_body(e, carry):
        def row_body(r, acc):
            row = e * CTX + r
            return tuple(
                acc[j] + rows_v[row, pl.ds(j * LANES, LANES)]
                for j in range(EMB_VREGS)
            )

        acc0 = tuple(
            jnp.zeros((LANES,), jnp.float32) for _ in range(EMB_VREGS)
        )
        acc = lax.fori_loop(0, CTX, row_body, acc0)
        for j in range(EMB_VREGS):
            pooled_v[e, pl.ds(j * LANES, LANES)] = acc[j] * inv_ctx
        return carry

    lax.fori_loop(0, B_PER_W, elem_body, 0)
    pltpu.sync_copy(pooled_v, out_hbm.at[pl.ds(wid * B_PER_W, B_PER_W)])


_pool_sc = functools.partial(
    pl.kernel,
    out_type=jax.ShapeDtypeStruct((B, EMB), jnp.float32),
    mesh=plsc.VectorSubcoreMesh(
        core_axis_name="c", subcore_axis_name="s", num_cores=NC,
        num_subcores=NS,
    ),
    scratch_types=[
        pltpu.VMEM((IDX_PER_W,), jnp.int32),
        pltpu.VMEM((IDX_PER_W, EMB), jnp.float32),
        pltpu.VMEM((B_PER_W, EMB), jnp.float32),
        pltpu.SemaphoreType.DMA,
    ],
    compiler_params=pltpu.CompilerParams(use_tc_tiling_on_sc=False),
)(_pool_body)


N_BLK = 2048


def _mm_body(w_ref, p_ref, b_ref, o_ref):
    wb = w_ref[...].astype(jnp.bfloat16)
    pb = p_ref[...].astype(jnp.bfloat16)
    bias8 = jnp.broadcast_to(b_ref[...] * 0.125, (8, N_BLK)).astype(
        jnp.bfloat16
    )
    ones8 = jnp.ones((B, 8), jnp.bfloat16)
    o_ref[...] = lax.dot_general(
        jnp.concatenate([wb, bias8], axis=0),
        jnp.concatenate([pb, ones8], axis=1),
        (((0,), (1,)), ((), ())),
        preferred_element_type=jnp.float32,
    )


def _matmul_tc(pooled, W, brow):
    n_blocks = pl.cdiv(VOC, N_BLK)
    return pl.pallas_call(
        _mm_body,
        grid=(n_blocks,),
        in_specs=[
            pl.BlockSpec((EMB, N_BLK), lambda i: (0, i)),
            pl.BlockSpec((B, EMB), lambda i: (0, 0)),
            pl.BlockSpec((1, N_BLK), lambda i: (0, i)),
        ],
        out_specs=pl.BlockSpec((N_BLK, B), lambda i: (i, 0)),
        out_shape=jax.ShapeDtypeStruct((VOC, B), jnp.float32),
        compiler_params=pltpu.CompilerParams(
            dimension_semantics=("arbitrary",),
        ),
    )(W, pooled, brow)


@jax.jit
def kernel(inpt, table, W, b):
    idx_flat = inpt.astype(jnp.int32).reshape(N_IDX)
    pooled = _pool_sc(idx_flat, table)
    outT = _matmul_tc(pooled, W, b.reshape(1, VOC))
    return outT.T


# speedup vs baseline: 2.8227x; 1.0049x over previous
import functools

import jax
import jax.numpy as jnp
from jax import lax
from jax.experimental import pallas as pl
from jax.experimental.pallas import tpu as pltpu
from jax.experimental.pallas import tpu_sc as plsc

B = 1024
CTX = 50
EMB = 64
VOC = 100000

NC = 2
NS = 16
NW = NC * NS
N_IDX = B * CTX
B_PER_W = B // NW
IDX_PER_W = B_PER_W * CTX
MAX_DESC = 128
LANES = 16
EMB_VREGS = EMB // LANES


def _pool_body(idx_hbm, table_hbm, out_hbm, idx_v, rows_v, pooled_v, sem):
    wid = lax.axis_index("s") * NC + lax.axis_index("c")
    base = wid * IDX_PER_W
    pltpu.sync_copy(idx_hbm.at[pl.ds(base, IDX_PER_W)], idx_v)

    descs = []
    off = 0
    while off < IDX_PER_W:
        n = min(MAX_DESC, IDX_PER_W - off)
        descs.append(
            pltpu.async_copy(
                table_hbm.at[idx_v.at[pl.ds(off, n)]],
                rows_v.at[pl.ds(off, n), :],
                sem,
            )
        )
        off += n
    for d in descs:
        d.wait()

    inv_ctx = jnp.float32(1.0 / CTX)

    def elem_body(e, carry):
        def row_body(r, acc):
            row = e * CTX + r
            return tuple(
                acc[j] + rows_v[row, pl.ds(j * LANES, LANES)]
                for j in range(EMB_VREGS)
            )

        acc0 = tuple(
            jnp.zeros((LANES,), jnp.float32) for _ in range(EMB_VREGS)
        )
        acc = lax.fori_loop(0, CTX, row_body, acc0)
        for j in range(EMB_VREGS):
            pooled_v[e, pl.ds(j * LANES, LANES)] = acc[j] * inv_ctx
        return carry

    lax.fori_loop(0, B_PER_W, elem_body, 0)
    pltpu.sync_copy(pooled_v, out_hbm.at[pl.ds(wid * B_PER_W, B_PER_W)])


_pool_sc = functools.partial(
    pl.kernel,
    out_type=jax.ShapeDtypeStruct((B, EMB), jnp.float32),
    mesh=plsc.VectorSubcoreMesh(
        core_axis_name="c", subcore_axis_name="s", num_cores=NC,
        num_subcores=NS,
    ),
    scratch_types=[
        pltpu.VMEM((IDX_PER_W,), jnp.int32),
        pltpu.VMEM((IDX_PER_W, EMB), jnp.float32),
        pltpu.VMEM((B_PER_W, EMB), jnp.float32),
        pltpu.SemaphoreType.DMA,
    ],
    compiler_params=pltpu.CompilerParams(use_tc_tiling_on_sc=False),
)(_pool_body)


N_BLK = 4096


def _mm_body(w_ref, p_ref, b_ref, o_ref):
    wb = w_ref[...].astype(jnp.bfloat16)
    pb = p_ref[...].astype(jnp.bfloat16)
    bias8 = jnp.broadcast_to(b_ref[...] * 0.125, (8, N_BLK)).astype(
        jnp.bfloat16
    )
    ones8 = jnp.ones((B, 8), jnp.bfloat16)
    o_ref[...] = lax.dot_general(
        jnp.concatenate([wb, bias8], axis=0),
        jnp.concatenate([pb, ones8], axis=1),
        (((0,), (1,)), ((), ())),
        preferred_element_type=jnp.float32,
    )


def _matmul_tc(pooled, W, brow):
    n_blocks = pl.cdiv(VOC, N_BLK)
    return pl.pallas_call(
        _mm_body,
        grid=(n_blocks,),
        in_specs=[
            pl.BlockSpec((EMB, N_BLK), lambda i: (0, i)),
            pl.BlockSpec((B, EMB), lambda i: (0, 0)),
            pl.BlockSpec((1, N_BLK), lambda i: (0, i)),
        ],
        out_specs=pl.BlockSpec((N_BLK, B), lambda i: (i, 0)),
        out_shape=jax.ShapeDtypeStruct((VOC, B), jnp.float32),
        compiler_params=pltpu.CompilerParams(
            dimension_semantics=("arbitrary",),
        ),
    )(W, pooled, brow)


@jax.jit
def kernel(inpt, table, W, b):
    idx_flat = inpt.astype(jnp.int32).reshape(N_IDX)
    pooled = _pool_sc(idx_flat, table)
    outT = _matmul_tc(pooled, W, b.reshape(1, VOC))
    return outT.T
